# Initial kernel scaffold; baseline (speedup 1.0000x reference)
import jax
import jax.numpy as jnp
from jax.experimental import pallas as pl


def kernel(queries, keys, values, Wq, bq, Wk, bk, Wv, bv, Wo, bo):
    raise NotImplementedError("write your pallas kernel here")


# speedup vs baseline: 25.9533x; 25.9533x over previous
import functools
import math

import jax
import jax.numpy as jnp
from jax.experimental import pallas as pl
from jax.experimental.pallas import tpu as pltpu

B, L, D, H, TOPK = 2, 2048, 2048, 16, 8
DK = D // H
MT = 512

_HIGH = jax.lax.Precision.HIGHEST


def _head_mean_mat():
    d_iota = jax.lax.broadcasted_iota(jnp.int32, (D, H), 0)
    h_iota = jax.lax.broadcasted_iota(jnp.int32, (D, H), 1)
    return jnp.where(d_iota // DK == h_iota, 1.0 / DK, 0.0).astype(jnp.float32)


def _red_kernel(wq_ref, wk_ref, bq_ref, bk_ref,
                wqh_ref, wkh_ref, bqh_ref, bkh_ref):
    ones = _head_mean_mat()
    wqh_ref[...] = jnp.dot(wq_ref[...], ones, precision=_HIGH,
                           preferred_element_type=jnp.float32)
    wkh_ref[...] = jnp.dot(wk_ref[...], ones, precision=_HIGH,
                           preferred_element_type=jnp.float32)
    bqh_ref[...] = jnp.dot(bq_ref[...], ones, precision=_HIGH,
                           preferred_element_type=jnp.float32)
    bkh_ref[...] = jnp.dot(bk_ref[...], ones, precision=_HIGH,
                           preferred_element_type=jnp.float32)


def _means_kernel(q_ref, k_ref, wqh_ref, wkh_ref, bqh_ref, bkh_ref,
                  qm_ref, km_ref):
    q = q_ref[0]
    k = k_ref[0]
    qm_ref[0] = jnp.dot(q, wqh_ref[...], precision=_HIGH,
                        preferred_element_type=jnp.float32) + bqh_ref[...]
    km_ref[0] = jnp.dot(k, wkh_ref[...], precision=_HIGH,
                        preferred_element_type=jnp.float32) + bkh_ref[...]


def _spectrum_kernel(qm_ref, km_ref, c_ref, s_ref, pr_ref, pi_ref):
    qm = qm_ref[...]
    km = km_ref[...]
    C = c_ref[...]
    S = s_ref[...]
    dot = functools.partial(jnp.dot, precision=_HIGH,
                            preferred_element_type=jnp.float32)
    qC = dot(qm, C)
    qS = dot(qm, S)
    kC = dot(km, C)
    kS = dot(km, S)
    pr_ref[...] = qC * kC + qS * kS
    pi_ref[...] = qC * kS - qS * kC


def _idft_kernel(pr_ref, pi_ref, c_ref, s_ref, corr_ref):
    dot = functools.partial(jnp.dot, precision=_HIGH,
                            preferred_element_type=jnp.float32)
    corr_ref[...] = (dot(pr_ref[...], c_ref[...])
                     - dot(pi_ref[...], s_ref[...])) * (1.0 / L)


def _topk_kernel(corr_ref, delays_ref, w_ref):
    corr = corr_ref[...]
    iota = jax.lax.broadcasted_iota(jnp.int32, corr.shape, 1)
    cur = corr
    vals, idxs = [], []
    for _ in range(TOPK):
        mx = jnp.max(cur, axis=1, keepdims=True)
        idx = jnp.min(jnp.where(cur == mx, iota, L), axis=1, keepdims=True)
        vals.append(mx)
        idxs.append(idx)
        cur = jnp.where(iota == idx, -1e30, cur)
    V = jnp.concatenate(vals, axis=1)
    I = jnp.concatenate(idxs, axis=1)
    e = jnp.exp(V - V[:, 0:1])
    w = e / jnp.sum(e, axis=1, keepdims=True)
    delays_ref[...] = I
    w_ref[...] = w


def _roll_kernel(delays_ref, wts_ref, v_ref, out_ref, v2):
    b = pl.program_id(0)
    h = pl.program_id(1)
    bh = b * H + h
    vblk = v_ref[0]
    v2[0:L, :] = vblk
    v2[L:2 * L, :] = vblk
    acc = jnp.zeros((L, DK), jnp.float32)
    for k in range(TOPK):
        d = delays_ref[bh * TOPK + k]
        w = wts_ref[bh * TOPK + k]
        acc = acc + w * v2[pl.ds(L - d, L), :]
    out_ref[0] = acc


def _proj_kernel(a_ref, w_ref, b_ref, o_ref):
    o_ref[...] = jnp.dot(a_ref[...], w_ref[...],
                         preferred_element_type=jnp.float32) + b_ref[...]


def _proj(x2d, W, bias2d):
    return pl.pallas_call(
        _proj_kernel,
        grid=(B * L // MT,),
        in_specs=[
            pl.BlockSpec((MT, D), lambda m: (m, 0)),
            pl.BlockSpec((D, D), lambda m: (0, 0)),
            pl.BlockSpec((1, D), lambda m: (0, 0)),
        ],
        out_specs=pl.BlockSpec((MT, D), lambda m: (m, 0)),
        out_shape=jax.ShapeDtypeStruct((B * L, D), jnp.float32),
    )(x2d, W, bias2d)


def kernel(queries, keys, values, Wq, bq, Wk, bk, Wv, bv, Wo, bo):
    f32 = jnp.float32
    bq2, bk2 = bq.reshape(1, D), bk.reshape(1, D)

    t = jnp.arange(L, dtype=jnp.int32)
    p = (t[:, None] * t[None, :]) % L
    ang = p.astype(f32) * f32(2.0 * math.pi / L)
    Ctab = jnp.cos(ang)
    Stab = jnp.sin(ang)

    wqh, wkh, bqh, bkh = pl.pallas_call(
        _red_kernel,
        grid=(4,),
        in_specs=[
            pl.BlockSpec((D // 4, D), lambda i: (i, 0)),
            pl.BlockSpec((D // 4, D), lambda i: (i, 0)),
            pl.BlockSpec((1, D), lambda i: (0, 0)),
            pl.BlockSpec((1, D), lambda i: (0, 0)),
        ],
        out_specs=[
            pl.BlockSpec((D // 4, H), lambda i: (i, 0)),
            pl.BlockSpec((D // 4, H), lambda i: (i, 0)),
            pl.BlockSpec((1, H), lambda i: (0, 0)),
            pl.BlockSpec((1, H), lambda i: (0, 0)),
        ],
        out_shape=[
            jax.ShapeDtypeStruct((D, H), f32),
            jax.ShapeDtypeStruct((D, H), f32),
            jax.ShapeDtypeStruct((1, H), f32),
            jax.ShapeDtypeStruct((1, H), f32),
        ],
    )(Wq, Wk, bq2, bk2)

    qm, km = pl.pallas_call(
        _means_kernel,
        grid=(B, L // MT),
        in_specs=[
            pl.BlockSpec((1, MT, D), lambda b, m: (b, m, 0)),
            pl.BlockSpec((1, MT, D), lambda b, m: (b, m, 0)),
            pl.BlockSpec((D, H), lambda b, m: (0, 0)),
            pl.BlockSpec((D, H), lambda b, m: (0, 0)),
            pl.BlockSpec((1, H), lambda b, m: (0, 0)),
            pl.BlockSpec((1, H), lambda b, m: (0, 0)),
        ],
        out_specs=[
            pl.BlockSpec((1, MT, H), lambda b, m: (b, m, 0)),
            pl.BlockSpec((1, MT, H), lambda b, m: (b, m, 0)),
        ],
        out_shape=[
            jax.ShapeDtypeStruct((B, L, H), f32),
            jax.ShapeDtypeStruct((B, L, H), f32),
        ],
    )(queries, keys, wqh, wkh, bqh, bkh)

    qmT = qm.transpose(0, 2, 1).reshape(B * H, L)
    kmT = km.transpose(0, 2, 1).reshape(B * H, L)

    FT = 512
    pr, pi = pl.pallas_call(
        _spectrum_kernel,
        grid=(L // FT,),
        in_specs=[
            pl.BlockSpec((B * H, L), lambda i: (0, 0)),
            pl.BlockSpec((B * H, L), lambda i: (0, 0)),
            pl.BlockSpec((L, FT), lambda i: (0, i)),
            pl.BlockSpec((L, FT), lambda i: (0, i)),
        ],
        out_specs=[
            pl.BlockSpec((B * H, FT), lambda i: (0, i)),
            pl.BlockSpec((B * H, FT), lambda i: (0, i)),
        ],
        out_shape=[
            jax.ShapeDtypeStruct((B * H, L), f32),
            jax.ShapeDtypeStruct((B * H, L), f32),
        ],
    )(qmT, kmT, Ctab, Stab)

    corr = pl.pallas_call(
        _idft_kernel,
        grid=(L // FT,),
        in_specs=[
            pl.BlockSpec((B * H, L), lambda i: (0, 0)),
            pl.BlockSpec((B * H, L), lambda i: (0, 0)),
            pl.BlockSpec((L, FT), lambda i: (0, i)),
            pl.BlockSpec((L, FT), lambda i: (0, i)),
        ],
        out_specs=pl.BlockSpec((B * H, FT), lambda i: (0, i)),
        out_shape=jax.ShapeDtypeStruct((B * H, L), f32),
    )(pr, pi, Ctab, Stab)

    delays, wts = pl.pallas_call(
        _topk_kernel,
        in_specs=[pl.BlockSpec((B * H, L), lambda: (0, 0))],
        out_specs=[
            pl.BlockSpec((B * H, TOPK), lambda: (0, 0)),
            pl.BlockSpec((B * H, TOPK), lambda: (0, 0)),
        ],
        out_shape=[
            jax.ShapeDtypeStruct((B * H, TOPK), jnp.int32),
            jax.ShapeDtypeStruct((B * H, TOPK), f32),
        ],
    )(corr)

    v = _proj(values.reshape(B * L, D), Wv, bv.reshape(1, D)).reshape(B, L, D)

    ctx = pl.pallas_call(
        _roll_kernel,
        grid=(B, H),
        in_specs=[
            pl.BlockSpec(memory_space=pltpu.SMEM),
            pl.BlockSpec(memory_space=pltpu.SMEM),
            pl.BlockSpec((1, L, DK), lambda b, h: (b, 0, h)),
        ],
        out_specs=pl.BlockSpec((1, L, DK), lambda b, h: (b, 0, h)),
        out_shape=jax.ShapeDtypeStruct((B, L, D), f32),
        scratch_shapes=[pltpu.VMEM((2 * L, DK), f32)],
    )(delays.reshape(-1), wts.reshape(-1), v)

    out = _proj(ctx.reshape(B * L, D), Wo, bo.reshape(1, D))
    return out.reshape(B, L, D)
